# baseline (speedup 1.0000x reference)
import jax
import jax.numpy as jnp
from jax.experimental import pallas as pl
from jax.experimental.pallas import tpu as pltpu

DM = 1024
DS = 16384
NT = 128
KK = 64

BN = 1024
NB = DS // BN

KEY_TINY = 1
KEY_INF = 0x7F800000
KEY_TS = [(127 - 2) << 23, 127 << 23, (127 + 2) << 23, (127 + 4) << 23]
NPLANES = 1 + len(KEY_TS)


def _body(x_ref, bdec_ref, wenc_ref, benc_ref, wdec_ref,
          acts_ref, recon_ref, key_ref, cnt_ref, lo_ref, acc_ref):
    j = pl.program_id(0)

    @pl.when(j < NB)
    def _encode():
        xc = x_ref[...] - bdec_ref[...]
        pre = (
            jnp.dot(xc, wenc_ref[...], preferred_element_type=jnp.float32)
            + benc_ref[...]
        )
        bits = pltpu.bitcast(pre, jnp.int32)
        key = jnp.where(bits < 0, bits ^ 0x7FFFFFFF, bits)
        key_ref[:, pl.ds(j * BN, BN)] = key

        @pl.when(j == 0)
        def _():
            cnt_ref[...] = jnp.zeros_like(cnt_ref)

        for idx, kt in enumerate([KEY_TINY] + KEY_TS):
            cnt_ref[idx] += (key >= kt).astype(jnp.int32)

    @pl.when(j == NB - 1)
    def _select():
        cpos = jnp.sum(cnt_ref[0], axis=1, keepdims=True)
        cs = [
            jnp.sum(cnt_ref[idx + 1], axis=1, keepdims=True)
            for idx in range(len(KEY_TS))
        ]
        lo = jnp.full((NT, 1), KEY_TINY, jnp.int32)
        clo = cpos
        for kt, c in zip(KEY_TS, cs):
            sel = c >= KK
            lo = jnp.where(sel, kt, lo)
            clo = jnp.where(sel, c, clo)
        hi = jnp.full((NT, 1), KEY_INF, jnp.int32)
        chi = jnp.zeros((NT, 1), jnp.int32)
        for kt, c in zip(KEY_TS[::-1], cs[::-1]):
            sel = c < KK
            hi = jnp.where(sel, kt, hi)
            chi = jnp.where(sel, c, chi)
        hi = jnp.where(cpos < KK, lo, hi)

        keys = key_ref[...]

        def cond(carry):
            i, lo_, clo_, hi_, chi_ = carry
            live = jnp.logical_and(clo_ != KK, hi_ - lo_ > 1)
            return jnp.logical_and(i < 48, jnp.any(live))

        def body(carry):
            i, lo_, clo_, hi_, chi_ = carry
            midb = (lo_ >> 1) + (hi_ >> 1) + (lo_ & hi_ & 1)
            frac = (clo_ - KK).astype(jnp.float32) / (
                (clo_ - chi_).astype(jnp.float32)
            )
            interp = lo_ + (
                (hi_ - lo_).astype(jnp.float32) * frac
            ).astype(jnp.int32)
            mid = jnp.where(i < 12, interp, midb)
            mid = jnp.clip(mid, lo_ + 1, hi_ - 1)
            cnt = jnp.sum((keys >= mid).astype(jnp.int32), axis=1,
                          keepdims=True)
            ok = cnt >= KK
            done = jnp.logical_or(clo_ == KK, hi_ - lo_ <= 1)
            lo2 = jnp.where(jnp.logical_or(done, ~ok), lo_, mid)
            clo2 = jnp.where(jnp.logical_or(done, ~ok), clo_, cnt)
            hi2 = jnp.where(jnp.logical_or(done, ok), hi_, mid)
            chi2 = jnp.where(jnp.logical_or(done, ok), chi_, cnt)
            return i + 1, lo2, clo2, hi2, chi2

        _, lo, _, _, _ = jax.lax.while_loop(
            cond, body, (jnp.int32(0), lo, clo, hi, chi)
        )
        lo_ref[...] = lo

    @pl.when(j >= NB)
    def _decode():
        jd = j - NB
        key = key_ref[:, pl.ds(jd * BN, BN)]
        a = jnp.where(
            key >= lo_ref[...], pltpu.bitcast(key, jnp.float32), 0.0
        )
        acts_ref[...] = a

        @pl.when(j == NB)
        def _():
            acc_ref[...] = jnp.zeros_like(acc_ref)

        acc_ref[...] += jnp.dot(
            a, wdec_ref[...], preferred_element_type=jnp.float32
        )

        @pl.when(j == 2 * NB - 1)
        def _():
            recon_ref[...] = acc_ref[...] + bdec_ref[...]


def kernel(x, W_enc, b_enc, W_dec, b_dec):
    b_enc2 = b_enc.reshape(1, DS)
    b_dec2 = b_dec.reshape(1, DM)

    acts, recon = pl.pallas_call(
        _body,
        grid=(2 * NB,),
        in_specs=[
            pl.BlockSpec((NT, DM), lambda j: (0, 0)),
            pl.BlockSpec((1, DM), lambda j: (0, 0)),
            pl.BlockSpec((DM, BN), lambda j: (0, jnp.minimum(j, NB - 1))),
            pl.BlockSpec((1, BN), lambda j: (0, jnp.minimum(j, NB - 1))),
            pl.BlockSpec(
                (BN, DM), lambda j: (jnp.maximum(j - NB, 0), 0)
            ),
        ],
        out_specs=[
            pl.BlockSpec(
                (NT, BN), lambda j: (0, jnp.maximum(j - NB, 0))
            ),
            pl.BlockSpec((NT, DM), lambda j: (0, 0)),
        ],
        out_shape=[
            jax.ShapeDtypeStruct((NT, DS), jnp.float32),
            jax.ShapeDtypeStruct((NT, DM), jnp.float32),
        ],
        scratch_shapes=[
            pltpu.VMEM((NT, DS), jnp.int32),
            pltpu.VMEM((NPLANES, NT, BN), jnp.int32),
            pltpu.VMEM((NT, 1), jnp.int32),
            pltpu.VMEM((NT, DM), jnp.float32),
        ],
    )(x, b_dec2, W_enc, b_enc2, W_dec)

    return (recon, acts)


# speedup vs baseline: 1.0586x; 1.0586x over previous
import jax
import jax.numpy as jnp
from jax.experimental import pallas as pl
from jax.experimental.pallas import tpu as pltpu

DM = 1024
DS = 16384
NT = 128
KK = 64

BN = 1024
NB = DS // BN

KEY_TINY = 1
KEY_INF = 0x7F800000
KEY_TS = [(127 + e) << 23 for e in range(-2, 5)]
NPLANES = 1 + len(KEY_TS)


def _body(x_ref, bdec_ref, wenc_ref, benc_ref, wdec_ref,
          acts_ref, recon_ref, key_ref, cnt_ref, lo_ref, acc_ref):
    j = pl.program_id(0)

    @pl.when(j < NB)
    def _encode():
        xc = x_ref[...] - bdec_ref[...]
        pre = (
            jnp.dot(xc, wenc_ref[...], preferred_element_type=jnp.float32)
            + benc_ref[...]
        )
        bits = pltpu.bitcast(pre, jnp.int32)
        key = jnp.where(bits < 0, bits ^ 0x7FFFFFFF, bits)
        key_ref[:, pl.ds(j * BN, BN)] = key

        @pl.when(j == 0)
        def _():
            cnt_ref[...] = jnp.zeros_like(cnt_ref)

        for idx, kt in enumerate([KEY_TINY] + KEY_TS):
            cnt_ref[idx] += (key >= kt).astype(jnp.int32)

    @pl.when(j == NB - 1)
    def _select():
        lo = jnp.full((NT, 1), KEY_TINY, jnp.int32)
        hi = jnp.full((NT, 1), KEY_INF, jnp.int32)
        cpos = jnp.sum(cnt_ref[0], axis=1, keepdims=True)
        hi = jnp.where(cpos < KK, lo, hi)
        for idx, kt in enumerate(KEY_TS):
            c = jnp.sum(cnt_ref[idx + 1], axis=1, keepdims=True)
            lo = jnp.where(c >= KK, kt, lo)
            hi = jnp.where(c < KK, jnp.minimum(hi, kt), hi)

        keys = key_ref[...]

        def cond(carry):
            i, lo_, hi_ = carry
            return jnp.logical_and(i < 34, jnp.any(hi_ - lo_ > 1))

        def body(carry):
            i, lo_, hi_ = carry
            mid = (lo_ >> 1) + (hi_ >> 1) + (lo_ & hi_ & 1)
            cnt = jnp.sum((keys >= mid).astype(jnp.int32), axis=1,
                          keepdims=True)
            ok = cnt >= KK
            return (
                i + 1,
                jnp.where(ok, mid, lo_),
                jnp.where(ok, hi_, mid),
            )

        _, lo, _ = jax.lax.while_loop(
            cond, body, (jnp.int32(0), lo, hi)
        )
        lo_ref[...] = lo

    @pl.when(j >= NB)
    def _decode():
        jd = j - NB
        key = key_ref[:, pl.ds(jd * BN, BN)]
        a = jnp.where(
            key >= lo_ref[...], pltpu.bitcast(key, jnp.float32), 0.0
        )
        acts_ref[...] = a

        @pl.when(j == NB)
        def _():
            acc_ref[...] = jnp.zeros_like(acc_ref)

        acc_ref[...] += jnp.dot(
            a, wdec_ref[...], preferred_element_type=jnp.float32
        )

        @pl.when(j == 2 * NB - 1)
        def _():
            recon_ref[...] = acc_ref[...] + bdec_ref[...]


def kernel(x, W_enc, b_enc, W_dec, b_dec):
    b_enc2 = b_enc.reshape(1, DS)
    b_dec2 = b_dec.reshape(1, DM)

    acts, recon = pl.pallas_call(
        _body,
        grid=(2 * NB,),
        in_specs=[
            pl.BlockSpec((NT, DM), lambda j: (0, 0)),
            pl.BlockSpec((1, DM), lambda j: (0, 0)),
            pl.BlockSpec((DM, BN), lambda j: (0, jnp.minimum(j, NB - 1))),
            pl.BlockSpec((1, BN), lambda j: (0, jnp.minimum(j, NB - 1))),
            pl.BlockSpec(
                (BN, DM), lambda j: (jnp.maximum(j - NB, 0), 0)
            ),
        ],
        out_specs=[
            pl.BlockSpec(
                (NT, BN), lambda j: (0, jnp.maximum(j - NB, 0))
            ),
            pl.BlockSpec((NT, DM), lambda j: (0, 0)),
        ],
        out_shape=[
            jax.ShapeDtypeStruct((NT, DS), jnp.float32),
            jax.ShapeDtypeStruct((NT, DM), jnp.float32),
        ],
        scratch_shapes=[
            pltpu.VMEM((NT, DS), jnp.int32),
            pltpu.VMEM((NPLANES, NT, BN), jnp.int32),
            pltpu.VMEM((NT, 1), jnp.int32),
            pltpu.VMEM((NT, DM), jnp.float32),
        ],
    )(x, b_dec2, W_enc, b_enc2, W_dec)

    return (recon, acts)


# speedup vs baseline: 1.0660x; 1.0069x over previous
import jax
import jax.numpy as jnp
from jax.experimental import pallas as pl
from jax.experimental.pallas import tpu as pltpu

DM = 1024
DS = 16384
NT = 128
KK = 64

BN = 1024
NB = DS // BN

KEY_TINY = 1
KEY_INF = 0x7F800000
KEY_TS = [(127 - 2) << 23, 127 << 23, (127 + 2) << 23, (127 + 4) << 23]
NPLANES = 1 + len(KEY_TS)


def _body(x_ref, bdec_ref, wenc_ref, benc_ref, wdec_ref,
          acts_ref, recon_ref, key_ref, cnt_ref, lo_ref, acc_ref):
    j = pl.program_id(0)

    @pl.when(j < NB)
    def _encode():
        xc = x_ref[...] - bdec_ref[...]
        pre = (
            jnp.dot(xc, wenc_ref[...], preferred_element_type=jnp.float32)
            + benc_ref[...]
        )
        bits = pltpu.bitcast(pre, jnp.int32)
        key = jnp.where(bits < 0, bits ^ 0x7FFFFFFF, bits)
        key_ref[:, pl.ds(j * BN, BN)] = key

        @pl.when(j == 0)
        def _():
            cnt_ref[...] = jnp.zeros_like(cnt_ref)

        for idx, kt in enumerate([KEY_TINY] + KEY_TS):
            cnt_ref[idx] += (key >= kt).astype(jnp.int32)

    @pl.when(j == NB - 1)
    def _select():
        lo = jnp.full((NT, 1), KEY_TINY, jnp.int32)
        hi = jnp.full((NT, 1), KEY_INF, jnp.int32)
        cpos = jnp.sum(cnt_ref[0], axis=1, keepdims=True)
        hi = jnp.where(cpos < KK, lo, hi)
        for idx, kt in enumerate(KEY_TS):
            c = jnp.sum(cnt_ref[idx + 1], axis=1, keepdims=True)
            lo = jnp.where(c >= KK, kt, lo)
            hi = jnp.where(c < KK, jnp.minimum(hi, kt), hi)

        keys = key_ref[...]

        def cond(carry):
            i, lo_, hi_ = carry
            return jnp.logical_and(i < 34, jnp.any(hi_ - lo_ > 1))

        def body(carry):
            i, lo_, hi_ = carry
            mid = (lo_ >> 1) + (hi_ >> 1) + (lo_ & hi_ & 1)
            cnt = jnp.sum((keys >= mid).astype(jnp.int32), axis=1,
                          keepdims=True)
            ok = cnt >= KK
            return (
                i + 1,
                jnp.where(ok, mid, lo_),
                jnp.where(ok, hi_, mid),
            )

        _, lo, _ = jax.lax.while_loop(
            cond, body, (jnp.int32(0), lo, hi)
        )
        lo_ref[...] = lo

    @pl.when(j >= NB)
    def _decode():
        jd = j - NB
        key = key_ref[:, pl.ds(jd * BN, BN)]
        a = jnp.where(
            key >= lo_ref[...], pltpu.bitcast(key, jnp.float32), 0.0
        )
        acts_ref[...] = a

        @pl.when(j == NB)
        def _():
            acc_ref[...] = jnp.zeros_like(acc_ref)

        acc_ref[...] += jnp.dot(
            a, wdec_ref[...], preferred_element_type=jnp.float32
        )

        @pl.when(j == 2 * NB - 1)
        def _():
            recon_ref[...] = acc_ref[...] + bdec_ref[...]


def kernel(x, W_enc, b_enc, W_dec, b_dec):
    b_enc2 = b_enc.reshape(1, DS)
    b_dec2 = b_dec.reshape(1, DM)

    acts, recon = pl.pallas_call(
        _body,
        grid=(2 * NB,),
        in_specs=[
            pl.BlockSpec((NT, DM), lambda j: (0, 0)),
            pl.BlockSpec((1, DM), lambda j: (0, 0)),
            pl.BlockSpec((DM, BN), lambda j: (0, jnp.minimum(j, NB - 1))),
            pl.BlockSpec((1, BN), lambda j: (0, jnp.minimum(j, NB - 1))),
            pl.BlockSpec(
                (BN, DM), lambda j: (jnp.maximum(j - NB, 0), 0)
            ),
        ],
        out_specs=[
            pl.BlockSpec(
                (NT, BN), lambda j: (0, jnp.maximum(j - NB, 0))
            ),
            pl.BlockSpec((NT, DM), lambda j: (0, 0)),
        ],
        out_shape=[
            jax.ShapeDtypeStruct((NT, DS), jnp.float32),
            jax.ShapeDtypeStruct((NT, DM), jnp.float32),
        ],
        scratch_shapes=[
            pltpu.VMEM((NT, DS), jnp.int32),
            pltpu.VMEM((NPLANES, NT, BN), jnp.int32),
            pltpu.VMEM((NT, 1), jnp.int32),
            pltpu.VMEM((NT, DM), jnp.float32),
        ],
    )(x, b_dec2, W_enc, b_enc2, W_dec)

    return (recon, acts)
